# baseline (speedup 1.0000x reference)
import jax
import jax.numpy as jnp
from jax import lax
from jax.experimental import pallas as pl
from jax.experimental.pallas import tpu as pltpu
from jax.experimental.pallas import tpu_sc as plsc

D = 64
SEQ = 200
BATCH = 4096
NC = 2
NS = 16
NW = NC * NS
EPS = 1e-12

_DN1 = lax.GatherDimensionNumbers(
    offset_dims=(), collapsed_slice_dims=(0,), start_index_map=(0,))


def _perm(v, p):
    return lax.gather(v, p[:, None], dimension_numbers=_DN1, slice_sizes=(1,),
                      mode=lax.GatherScatterMode.PROMISE_IN_BOUNDS)


def _lane_sum(v, perms):
    for p in perms:
        v = v + _perm(v, p)
    return v


def _rsqrt(x):
    i = lax.bitcast_convert_type(x, jnp.int32)
    i = jnp.int32(0x5F3759DF) - (i >> 1)
    y = lax.bitcast_convert_type(i, jnp.float32)
    for _ in range(2):
        y = y * (1.5 - 0.5 * x * y * y)
    return y


def _body(x_hbm, w_hbm, p_hbm, gb_hbm, out_hbm,
          p_v, gb_v, idx_a, idx_b, idx2_a, idx2_b, fb_a, fb_b,
          rows_a, rows_b, out_a, out_b, sem_ga, sem_gb, sem_oa, sem_ob):
    idx_ = (idx_a, idx_b)
    idx2_ = (idx2_a, idx2_b)
    fb_ = (fb_a, fb_b)
    rows_ = (rows_a, rows_b)
    outv_ = (out_a, out_b)
    semg_ = (sem_ga, sem_gb)
    semo_ = (sem_oa, sem_ob)
    wid = lax.axis_index("s") * NC + lax.axis_index("c")

    pltpu.sync_copy(p_hbm, p_v)
    pltpu.sync_copy(gb_hbm, gb_v)

    g03 = [gb_v[pl.ds(16 * k, 16)] for k in range(4)]
    b03 = [gb_v[pl.ds(D + 16 * k, 16)] for k in range(4)]

    iota = lax.iota(jnp.int32, 16)
    perms = [iota ^ k for k in (1, 2, 4, 8)]
    d_k = [iota + 16 * k for k in range(4)]
    lane = iota & 15

    def start_gather(buf, l):
        pltpu.sync_copy(x_hbm.at[l, wid], idx_[buf])
        for k in range(8):
            chunk = idx_[buf][pl.ds(16 * k, 16)]
            idx2_[buf][pl.ds(16 * k, 16)] = chunk >> 1
            fb_[buf][k] = (chunk & 1) * D
        pltpu.async_copy(w_hbm.at[idx2_[buf]], rows_[buf], semg_[buf])

    def wait_gather(buf):
        pltpu.make_async_copy(w_hbm.at[idx2_[buf]], rows_[buf],
                              semg_[buf]).wait()

    def compute(buf, l):
        lh = l >> 1
        lo = (l & 1) * D
        p03 = [p_v[lh, pl.ds(lo + 16 * k, 16)] for k in range(4)]

        @plsc.parallel_loop(0, 128, unroll=4)
        def row_body(r):
            rb = jnp.zeros((16,), jnp.int32) + r
            blk = fb_[buf][r >> 4, :]
            offv = _perm(blk, rb & 15)
            rf = rows_[buf]
            e0 = plsc.load_gather(rf, [rb, offv + d_k[0]]) + p03[0]
            e1 = plsc.load_gather(rf, [rb, offv + d_k[1]]) + p03[1]
            e2 = plsc.load_gather(rf, [rb, offv + d_k[2]]) + p03[2]
            e3 = plsc.load_gather(rf, [rb, offv + d_k[3]]) + p03[3]
            tot = _lane_sum(e0 + e1 + e2 + e3, perms)
            tot2 = _lane_sum(e0 * e0 + e1 * e1 + e2 * e2 + e3 * e3, perms)
            mean = tot * (1.0 / D)
            var = tot2 * (1.0 / D) - mean * mean
            inv = _rsqrt(var + EPS)
            shift = -mean * inv
            es = (e0, e1, e2, e3)
            for k in range(4):
                o = (es[k] * inv + shift) * g03[k] + b03[k]
                plsc.store_scatter(outv_[buf], [d_k[k], rb], o)

    def out_dma(buf, l):
        pltpu.async_copy(outv_[buf],
                         out_hbm.at[l, :, pl.ds(128 * wid, 128)], semo_[buf])

    def wait_out(buf, l):
        pltpu.make_async_copy(outv_[buf],
                              out_hbm.at[l, :, pl.ds(128 * wid, 128)],
                              semo_[buf]).wait()

    start_gather(0, 0)

    def outer(i, _):
        for b in (0, 1):
            l = 2 * i + b
            start_gather(1 - b, jnp.minimum(l + 1, SEQ - 1))
            wait_gather(b)

            @pl.when(i >= 1)
            def _():
                wait_out(b, jnp.maximum(l - 2, 0))

            compute(b, l)
            out_dma(b, l)
        return 0

    lax.fori_loop(0, SEQ // 2, outer, 0)
    wait_gather(0)
    wait_out(0, SEQ - 2)
    wait_out(1, SEQ - 1)


def kernel(x, W, P, gamma, beta):
    x3 = x.T.reshape(SEQ, NW, 128).astype(jnp.int32)
    w2 = W.reshape(500000, 128)
    p2 = P.reshape(SEQ // 2, 128)
    gb = jnp.concatenate([gamma, beta])
    mesh = plsc.VectorSubcoreMesh(core_axis_name="c", subcore_axis_name="s")
    run = pl.kernel(
        _body,
        out_type=jax.ShapeDtypeStruct((SEQ, D, BATCH), jnp.float32),
        mesh=mesh,
        compiler_params=pltpu.CompilerParams(use_tc_tiling_on_sc=False,
                                             needs_layout_passes=False),
        scratch_types=[
            pltpu.VMEM((SEQ // 2, 128), jnp.float32),
            pltpu.VMEM((2 * D,), jnp.float32),
            pltpu.VMEM((128,), jnp.int32),
            pltpu.VMEM((128,), jnp.int32),
            pltpu.VMEM((128,), jnp.int32),
            pltpu.VMEM((128,), jnp.int32),
            pltpu.VMEM((8, 16), jnp.int32),
            pltpu.VMEM((8, 16), jnp.int32),
            pltpu.VMEM((128, 128), jnp.float32),
            pltpu.VMEM((128, 128), jnp.float32),
            pltpu.VMEM((D, 128), jnp.float32),
            pltpu.VMEM((D, 128), jnp.float32),
            pltpu.SemaphoreType.DMA,
            pltpu.SemaphoreType.DMA,
            pltpu.SemaphoreType.DMA,
            pltpu.SemaphoreType.DMA,
        ],
    )
    out3 = run(x3, w2, p2, gb)
    return out3.transpose(2, 0, 1)


# speedup vs baseline: 1.0318x; 1.0318x over previous
import jax
import jax.numpy as jnp
from jax import lax
from jax.experimental import pallas as pl
from jax.experimental.pallas import tpu as pltpu
from jax.experimental.pallas import tpu_sc as plsc

D = 64
SEQ = 200
BATCH = 4096
NC = 2
NS = 16
NW = NC * NS
EPS = 1e-12

_DN1 = lax.GatherDimensionNumbers(
    offset_dims=(), collapsed_slice_dims=(0,), start_index_map=(0,))


def _perm(v, p):
    return lax.gather(v, p[:, None], dimension_numbers=_DN1, slice_sizes=(1,),
                      mode=lax.GatherScatterMode.PROMISE_IN_BOUNDS)


def _lane_sum(v, perms):
    for p in perms:
        v = v + _perm(v, p)
    return v


def _rsqrt(x):
    i = lax.bitcast_convert_type(x, jnp.int32)
    i = jnp.int32(0x5F3759DF) - (i >> 1)
    y = lax.bitcast_convert_type(i, jnp.float32)
    for _ in range(2):
        y = y * (1.5 - 0.5 * x * y * y)
    return y


def _body(x_hbm, w_hbm, p_hbm, gb_hbm, out_hbm,
          p_v, gb_v, idx_a, idx_b,
          rows_a, rows_b, out_a, out_b, sem_ga, sem_gb, sem_oa, sem_ob):
    idx_ = (idx_a, idx_b)
    rows_ = (rows_a, rows_b)
    outv_ = (out_a, out_b)
    semg_ = (sem_ga, sem_gb)
    semo_ = (sem_oa, sem_ob)
    wid = lax.axis_index("s") * NC + lax.axis_index("c")

    pltpu.sync_copy(p_hbm, p_v)
    pltpu.sync_copy(gb_hbm, gb_v)

    g03 = [gb_v[pl.ds(16 * k, 16)] for k in range(4)]
    b03 = [gb_v[pl.ds(D + 16 * k, 16)] for k in range(4)]

    iota = lax.iota(jnp.int32, 16)
    perms = [iota ^ k for k in (1, 2, 4, 8)]
    d_k = [iota + 16 * k for k in range(4)]
    lane = iota & 15

    def start_gather(buf, l):
        pltpu.sync_copy(x_hbm.at[l, wid], idx_[buf])
        pltpu.async_copy(w_hbm.at[idx_[buf]], rows_[buf], semg_[buf])

    def wait_gather(buf):
        pltpu.make_async_copy(w_hbm.at[idx_[buf]], rows_[buf],
                              semg_[buf]).wait()

    def compute(buf, l):
        lh = l >> 1
        lo = (l & 1) * D
        p03 = [p_v[lh, pl.ds(lo + 16 * k, 16)] for k in range(4)]

        @plsc.parallel_loop(0, 128, unroll=4)
        def row_body(r):
            rb = jnp.zeros((16,), jnp.int32) + r
            rf = rows_[buf]
            e0 = rf[r, pl.ds(0, 16)] + p03[0]
            e1 = rf[r, pl.ds(16, 16)] + p03[1]
            e2 = rf[r, pl.ds(32, 16)] + p03[2]
            e3 = rf[r, pl.ds(48, 16)] + p03[3]
            tot = _lane_sum(e0 + e1 + e2 + e3, perms)
            tot2 = _lane_sum(e0 * e0 + e1 * e1 + e2 * e2 + e3 * e3, perms)
            mean = tot * (1.0 / D)
            var = tot2 * (1.0 / D) - mean * mean
            inv = _rsqrt(var + EPS)
            shift = -mean * inv
            es = (e0, e1, e2, e3)
            for k in range(4):
                o = (es[k] * inv + shift) * g03[k] + b03[k]
                plsc.store_scatter(outv_[buf], [d_k[k], rb], o)

    def out_dma(buf, l):
        pltpu.async_copy(outv_[buf],
                         out_hbm.at[l, :, pl.ds(128 * wid, 128)], semo_[buf])

    def wait_out(buf, l):
        pltpu.make_async_copy(outv_[buf],
                              out_hbm.at[l, :, pl.ds(128 * wid, 128)],
                              semo_[buf]).wait()

    start_gather(0, 0)

    def outer(i, _):
        for b in (0, 1):
            l = 2 * i + b
            start_gather(1 - b, jnp.minimum(l + 1, SEQ - 1))
            wait_gather(b)

            @pl.when(i >= 1)
            def _():
                wait_out(b, jnp.maximum(l - 2, 0))

            compute(b, l)
            out_dma(b, l)
        return 0

    lax.fori_loop(0, SEQ // 2, outer, 0)
    wait_gather(0)
    wait_out(0, SEQ - 2)
    wait_out(1, SEQ - 1)


def kernel(x, W, P, gamma, beta):
    x3 = x.T.reshape(SEQ, NW, 128).astype(jnp.int32)
    p2 = P.reshape(SEQ // 2, 128)
    gb = jnp.concatenate([gamma, beta])
    mesh = plsc.VectorSubcoreMesh(core_axis_name="c", subcore_axis_name="s")
    run = pl.kernel(
        _body,
        out_type=jax.ShapeDtypeStruct((SEQ, D, BATCH), jnp.float32),
        mesh=mesh,
        compiler_params=pltpu.CompilerParams(use_tc_tiling_on_sc=False,
                                             needs_layout_passes=False),
        scratch_types=[
            pltpu.VMEM((SEQ // 2, 128), jnp.float32),
            pltpu.VMEM((2 * D,), jnp.float32),
            pltpu.VMEM((128,), jnp.int32),
            pltpu.VMEM((128,), jnp.int32),
            pltpu.VMEM((128, D), jnp.float32),
            pltpu.VMEM((128, D), jnp.float32),
            pltpu.VMEM((D, 128), jnp.float32),
            pltpu.VMEM((D, 128), jnp.float32),
            pltpu.SemaphoreType.DMA,
            pltpu.SemaphoreType.DMA,
            pltpu.SemaphoreType.DMA,
            pltpu.SemaphoreType.DMA,
        ],
    )
    out3 = run(x3, W, p2, gb)
    return out3.transpose(2, 0, 1)


# speedup vs baseline: 1.3258x; 1.2849x over previous
import jax
import jax.numpy as jnp
from jax import lax
from jax.experimental import pallas as pl
from jax.experimental.pallas import tpu as pltpu
from jax.experimental.pallas import tpu_sc as plsc

D = 64
SEQ = 200
BATCH = 4096
NC = 2
NS = 16
NW = NC * NS
SEQ_PER_W = BATCH // NW
EPS = 1e-12


def _lane_sum(v, perms):
    dn = lax.GatherDimensionNumbers(
        offset_dims=(), collapsed_slice_dims=(0,), start_index_map=(0,))
    for p in perms:
        v = v + lax.gather(v, p[:, None], dimension_numbers=dn,
                           slice_sizes=(1,),
                           mode=lax.GatherScatterMode.PROMISE_IN_BOUNDS)
    return v


def _rsqrt(x):
    i = lax.bitcast_convert_type(x, jnp.int32)
    i = jnp.int32(0x5F3759DF) - (i >> 1)
    y = lax.bitcast_convert_type(i, jnp.float32)
    for _ in range(2):
        y = y * (1.5 - 0.5 * x * y * y)
    return y


def _body(x_hbm, w_hbm, p_hbm, g_hbm, b_hbm, out_hbm,
          p_v, g_v, b_v, idx_v, rows_v, out_v, sem_g, sem_o):
    wid = lax.axis_index("s") * NC + lax.axis_index("c")

    pltpu.sync_copy(p_hbm, p_v)
    pltpu.sync_copy(g_hbm, g_v)
    pltpu.sync_copy(b_hbm, b_v)

    g03 = [g_v[pl.ds(16 * k, 16)] for k in range(4)]
    b03 = [b_v[pl.ds(16 * k, 16)] for k in range(4)]

    iota = lax.iota(jnp.int32, 16)
    perms = [iota ^ k for k in (1, 2, 4, 8)]

    def start_gather(b, seq):
        pltpu.sync_copy(x_hbm.at[seq], idx_v.at[b])
        pltpu.async_copy(w_hbm.at[idx_v.at[b, 0]], rows_v.at[b, pl.ds(0, 100)],
                         sem_g.at[b])
        pltpu.async_copy(w_hbm.at[idx_v.at[b, 1]], rows_v.at[b, pl.ds(100, 100)],
                         sem_g.at[b])

    def wait_gather(b):
        pltpu.make_async_copy(w_hbm.at[idx_v.at[b, 0]],
                              rows_v.at[b, pl.ds(0, 100)], sem_g.at[b]).wait()
        pltpu.make_async_copy(w_hbm.at[idx_v.at[b, 1]],
                              rows_v.at[b, pl.ds(100, 100)], sem_g.at[b]).wait()

    def wait_out(b, seq):
        pltpu.make_async_copy(out_v.at[b], out_hbm.at[seq], sem_o.at[b]).wait()

    def compute(b):
        @plsc.parallel_loop(0, SEQ, unroll=8)
        def row_body(r):
            e0 = rows_v[b, r, pl.ds(0, 16)] + p_v[r, pl.ds(0, 16)]
            e1 = rows_v[b, r, pl.ds(16, 16)] + p_v[r, pl.ds(16, 16)]
            e2 = rows_v[b, r, pl.ds(32, 16)] + p_v[r, pl.ds(32, 16)]
            e3 = rows_v[b, r, pl.ds(48, 16)] + p_v[r, pl.ds(48, 16)]
            tot = _lane_sum(e0 + e1 + e2 + e3, perms)
            tot2 = _lane_sum(e0 * e0 + e1 * e1 + e2 * e2 + e3 * e3, perms)
            mean = tot * (1.0 / D)
            var = tot2 * (1.0 / D) - mean * mean
            inv = _rsqrt(var + EPS)
            shift = -mean * inv
            out_v[b, r, pl.ds(0, 16)] = (e0 * inv + shift) * g03[0] + b03[0]
            out_v[b, r, pl.ds(16, 16)] = (e1 * inv + shift) * g03[1] + b03[1]
            out_v[b, r, pl.ds(32, 16)] = (e2 * inv + shift) * g03[2] + b03[2]
            out_v[b, r, pl.ds(48, 16)] = (e3 * inv + shift) * g03[3] + b03[3]

    seq0 = wid * SEQ_PER_W
    start_gather(0, seq0)

    def outer(i, _):
        for b in (0, 1):
            s = 2 * i + b
            seq = seq0 + s
            start_gather(1 - b, jnp.minimum(seq + 1, BATCH - 1))
            wait_gather(b)

            @pl.when(i >= 1)
            def _():
                wait_out(b, jnp.maximum(seq - 2, 0))

            compute(b)
            pltpu.async_copy(out_v.at[b], out_hbm.at[seq], sem_o.at[b])
        return 0

    lax.fori_loop(0, SEQ_PER_W // 2, outer, 0)
    wait_gather(0)
    wait_out(0, seq0 + SEQ_PER_W - 2)
    wait_out(1, seq0 + SEQ_PER_W - 1)


def kernel(x, W, P, gamma, beta):
    x2 = x.reshape(BATCH, 2, SEQ // 2).astype(jnp.int32)
    mesh = plsc.VectorSubcoreMesh(core_axis_name="c", subcore_axis_name="s")
    run = pl.kernel(
        _body,
        out_type=jax.ShapeDtypeStruct((BATCH, SEQ, D), jnp.float32),
        mesh=mesh,
        compiler_params=pltpu.CompilerParams(use_tc_tiling_on_sc=False),
        scratch_types=[
            pltpu.VMEM((SEQ, D), jnp.float32),
            pltpu.VMEM((D,), jnp.float32),
            pltpu.VMEM((D,), jnp.float32),
            pltpu.VMEM((2, 2, SEQ // 2), jnp.int32),
            pltpu.VMEM((2, SEQ, D), jnp.float32),
            pltpu.VMEM((2, SEQ, D), jnp.float32),
            pltpu.SemaphoreType.DMA((2,)),
            pltpu.SemaphoreType.DMA((2,)),
        ],
    )
    return run(x2, W, P, gamma, beta)
